# baseline (speedup 1.0000x reference)
import functools

import jax
import jax.numpy as jnp
from jax import lax
from jax.experimental import pallas as pl
from jax.experimental.pallas import tpu as pltpu
from jax.experimental.pallas import tpu_sc as plsc

B, D, L = 16, 512, 4096
LB = 512
CG = 16
DHALF = D // 2
NCH_GROUPS = DHALF // CG
LANES = 16
CHUNK = 8 * LANES


def _body(x_hbm, len_hbm, out_hbm, buf, acc, out_stage, len_v, sems):
    c = lax.axis_index("c")
    s = lax.axis_index("s")
    wid = s * 2 + c
    b = wid // 2
    half = wid % 2
    chbase = half * DHALF

    pltpu.sync_copy(len_hbm, len_v.at[pl.ds(0, B)])
    lane = jnp.arange(LANES, dtype=jnp.int32)
    len_b = len_v[pl.ds(b, LANES)][0]
    n_blocks = (len_b + (LB - 1)) // LB
    total_units = NCH_GROUPS * n_blocks

    inf_v = jnp.full((LANES,), jnp.inf, dtype=jnp.float32)

    def issue(u, g, blk):
        slot = u % 2
        ch0 = chbase + g * CG
        l0 = blk * LB
        pltpu.async_copy(
            x_hbm.at[b, pl.ds(ch0, CG), pl.ds(l0, LB)],
            buf.at[slot],
            sems.at[slot],
        )

    def wait(u, g, blk):
        slot = u % 2
        ch0 = chbase + g * CG
        l0 = blk * LB
        pltpu.make_async_copy(
            x_hbm.at[b, pl.ds(ch0, CG), pl.ds(l0, LB)],
            buf.at[slot],
            sems.at[slot],
        ).wait()

    def compute(u, g, blk):
        slot = u % 2
        l0 = blk * LB
        navail = jnp.minimum(LB, len_b - l0)
        n_chunks = navail // CHUNK
        rem = navail - n_chunks * CHUNK

        @pl.when(blk == 0)
        def _():
            def init_body(ch, carry):
                acc[ch] = inf_v
                return carry

            lax.fori_loop(0, CG, init_body, 0)

        def ch_body(ch, carry):
            a = acc[ch]

            def chunk_body(t, a2):
                base = t * CHUNK
                for jj in range(CHUNK // LANES):
                    v = buf[slot, ch, pl.ds(base + jj * LANES, LANES)]
                    a2 = jnp.minimum(a2, v)
                return a2

            a = lax.fori_loop(0, n_chunks, chunk_body, a)

            rbase = n_chunks * CHUNK
            for jj in range(CHUNK // LANES):
                off = jj * LANES
                v = buf[slot, ch, pl.ds(rbase + off, LANES)]
                v = jnp.where(lane < rem - off, v, inf_v)
                a = jnp.minimum(a, v)

            acc[ch] = a
            return carry

        lax.fori_loop(0, CG, ch_body, 0)

        @pl.when(blk == n_blocks - 1)
        def _():
            def pack_body(ch, res):
                m = acc[ch]
                for k in (8, 4, 2, 1):
                    perm = jnp.bitwise_xor(lane, k)
                    m = jnp.minimum(m, m.at[perm].get(mode="promise_in_bounds"))
                return jnp.where(lane == ch, m, res)

            res = lax.fori_loop(0, CG, pack_body, inf_v)
            out_stage[pl.ds(g * CG, CG)] = res

    issue(0, 0, 0)

    def unit_body(u, carry):
        g, blk = carry
        nb = blk + 1
        wrap = nb == n_blocks
        nblk = jnp.where(wrap, 0, nb)
        ng = g + wrap.astype(jnp.int32)

        @pl.when(u + 1 < total_units)
        def _():
            issue(u + 1, ng, nblk)

        wait(u, g, blk)
        compute(u, g, blk)
        return (ng, nblk)

    lax.fori_loop(0, total_units, unit_body,
                  (jnp.int32(0), jnp.int32(0)))

    pltpu.sync_copy(out_stage, out_hbm.at[b, pl.ds(chbase, DHALF)])


@functools.partial(
    pl.kernel,
    mesh=plsc.VectorSubcoreMesh(core_axis_name="c", subcore_axis_name="s"),
    out_type=jax.ShapeDtypeStruct((B, D), jnp.float32),
    scratch_types=[
        pltpu.VMEM((2, CG, LB), jnp.float32),
        pltpu.VMEM((CG, LANES), jnp.float32),
        pltpu.VMEM((DHALF,), jnp.float32),
        pltpu.VMEM((2 * B,), jnp.int32),
        pltpu.SemaphoreType.DMA((2,)),
    ],
)
def _pool_min(x_hbm, len_hbm, out_hbm, buf, acc, out_stage, len_v, sems):
    _body(x_hbm, len_hbm, out_hbm, buf, acc, out_stage, len_v, sems)


def kernel(x0, x1, x2):
    del x1
    return _pool_min(x0, x2)


# speedup vs baseline: 1.9272x; 1.9272x over previous
import functools

import jax
import jax.numpy as jnp
from jax import lax
from jax.experimental import pallas as pl
from jax.experimental.pallas import tpu as pltpu
from jax.experimental.pallas import tpu_sc as plsc

B, D, L = 16, 512, 4096
LB = 512
CG = 16
LANES = 16
NBUF = 4
CHUNK = 8 * LANES
LEN_PAD = 64


def _body(x_hbm, len_hbm, out_hbm, buf, acc, out_stage, len_v, sems, shared, tmp, fin):
    c = lax.axis_index("c")
    s = lax.axis_index("s")
    wid = c * 16 + s
    ch0 = wid * CG

    pltpu.sync_copy(len_hbm, len_v.at[pl.ds(0, B)])
    lane = jnp.arange(LANES, dtype=jnp.int32)
    inf_v = jnp.full((LANES,), jnp.inf, dtype=jnp.float32)

    def nblocks_of(b):
        ln = len_v[pl.ds(b, LANES)][0]
        return (ln + (LB - 1)) // LB, ln

    def total_body(b, t):
        nb, _ = nblocks_of(b)
        return t + nb

    total_units = lax.fori_loop(0, B, total_body, jnp.int32(0))

    def advance(st):
        b, blk, nb, ln = st
        nxt = blk + 1
        wrap = nxt == nb
        b2 = jnp.minimum(b + wrap.astype(jnp.int32), B - 1)
        blk2 = jnp.where(wrap, 0, nxt)
        nb2, ln2 = nblocks_of(b2)
        return (b2, blk2, jnp.where(wrap, nb2, nb), jnp.where(wrap, ln2, ln))

    def issue(u, st):
        b, blk, _, _ = st
        slot = u % NBUF
        pltpu.async_copy(
            x_hbm.at[b, pl.ds(ch0, CG), pl.ds(blk * LB, LB)],
            buf.at[slot],
            sems.at[slot],
        )

    def wait(u, st):
        b, blk, _, _ = st
        slot = u % NBUF
        pltpu.make_async_copy(
            x_hbm.at[b, pl.ds(ch0, CG), pl.ds(blk * LB, LB)],
            buf.at[slot],
            sems.at[slot],
        ).wait()

    def compute(u, st):
        b, blk, nb, ln = st
        slot = u % NBUF
        l0 = blk * LB
        navail = jnp.minimum(LB, ln - l0)
        n_chunks = navail // CHUNK
        rem = navail - n_chunks * CHUNK

        @pl.when(blk == 0)
        def _():
            def init_body(ch, carry):
                acc[ch] = inf_v
                return carry

            lax.fori_loop(0, CG, init_body, 0)

        def ch_body(ch, carry):
            a = acc[ch]

            def chunk_body(t, a2):
                base = t * CHUNK
                for jj in range(CHUNK // LANES):
                    v = buf[slot, ch, pl.ds(base + jj * LANES, LANES)]
                    a2 = jnp.minimum(a2, v)
                return a2

            a = lax.fori_loop(0, n_chunks, chunk_body, a)

            @pl.when(rem > 0)
            def _():
                a2 = a
                rbase = n_chunks * CHUNK
                for jj in range(CHUNK // LANES):
                    off = jj * LANES
                    v = buf[slot, ch, pl.ds(rbase + off, LANES)]
                    v = jnp.where(lane < rem - off, v, inf_v)
                    a2 = jnp.minimum(a2, v)
                acc[ch] = a2

            @pl.when(rem == 0)
            def _():
                acc[ch] = a

            return carry

        lax.fori_loop(0, CG, ch_body, 0)

        @pl.when(blk == nb - 1)
        def _():
            def pack_body(ch, res):
                m = acc[ch]
                for k in (8, 4, 2, 1):
                    perm = jnp.bitwise_xor(lane, k)
                    m = jnp.minimum(m, m.at[perm].get(mode="promise_in_bounds"))
                return jnp.where(lane == ch, m, res)

            out_stage[pl.ds(b * CG, CG)] = lax.fori_loop(0, CG, pack_body, inf_v)

    def pro_body(u, st):
        @pl.when(u < total_units)
        def _():
            issue(u, st)

        return advance(st)

    nb0, ln0 = nblocks_of(0)
    st0 = (jnp.int32(0), jnp.int32(0), nb0, ln0)
    ist = lax.fori_loop(0, NBUF - 1, pro_body, st0)

    def unit_body(u, carry):
        cst, ist = carry

        @pl.when(u + (NBUF - 1) < total_units)
        def _():
            issue(u + (NBUF - 1), ist)

        ist2 = advance(ist)
        wait(u, cst)
        compute(u, cst)
        return (advance(cst), ist2)

    lax.fori_loop(0, total_units, unit_body, (st0, ist))

    pltpu.sync_copy(out_stage, out_hbm.at[wid])


@functools.partial(
    pl.kernel,
    mesh=plsc.VectorSubcoreMesh(core_axis_name="c", subcore_axis_name="s"),
    out_type=jax.ShapeDtypeStruct((32, B * CG), jnp.float32),
    scratch_types=[
        pltpu.VMEM((NBUF, CG, LB), jnp.float32),
        pltpu.VMEM((CG, LANES), jnp.float32),
        pltpu.VMEM((B * CG,), jnp.float32),
        pltpu.VMEM((LEN_PAD,), jnp.int32),
        pltpu.SemaphoreType.DMA((NBUF,)),
        pltpu.VMEM_SHARED((16, B, CG), jnp.float32),
        pltpu.VMEM((16, B, CG), jnp.float32),
        pltpu.VMEM((B, 16 * CG), jnp.float32),
    ],
)
def _pool_min(x_hbm, len_hbm, out_hbm, buf, acc, out_stage, len_v, sems,
              shared, tmp, fin):
    _body(x_hbm, len_hbm, out_hbm, buf, acc, out_stage, len_v, sems, shared,
          tmp, fin)


def kernel(x0, x1, x2):
    del x1
    raw = _pool_min(x0, x2)
    return raw.reshape(32, B, CG).transpose(1, 0, 2).reshape(B, D)
